# baseline (speedup 1.0000x reference)
import functools

import jax
import jax.numpy as jnp
from jax import lax
from jax.experimental import pallas as pl
from jax.experimental.pallas import tpu as pltpu
from jax.experimental.pallas import tpu_sc as plsc

N = 10000
E = 320000
DIN = 128
DH = 64
DOUT = 40
DOUTP = 48

NC = 2
NS = 16
NW = NC * NS
C = 128

E2 = E + N
G0 = 96
G1 = 72
GMX = max(G0, G1)
NG_AGG = NS * (G0 + G1)
DEG_RPT = 88
NGT = NW * DEG_RPT
EPAD = NGT * C
NPAD = 10240
NPT = NPAD // NS
assert E2 <= NG_AGG * C and NS * G0 + (NS - 1) * G1 + GMX <= NGT
assert G0 % 24 == 0 and G1 % 24 == 0 and DEG_RPT % 8 == 0
assert NPT % C == 0 and N % 16 == 0

_mesh = plsc.VectorSubcoreMesh(core_axis_name="c", subcore_axis_name="s")
_sc_params = pltpu.CompilerParams(needs_layout_passes=False,
                                  use_tc_tiling_on_sc=False)


@functools.partial(
    pl.kernel,
    out_type=jax.ShapeDtypeStruct((NW, N), jnp.float32),
    mesh=_mesh,
    compiler_params=_sc_params,
    scratch_types=[
        pltpu.VMEM((DEG_RPT, C), jnp.int32),
        pltpu.VMEM((DEG_RPT, C), jnp.float32),
        pltpu.VMEM((N,), jnp.float32),
    ],
)
def _deg_sc(dst_hbm, ew_hbm, out_hbm, dst_v, ew_v, deg_v):
    c = lax.axis_index("c")
    s = lax.axis_index("s")
    w = c * NS + s
    pltpu.sync_copy(dst_hbm.at[pl.ds(w * DEG_RPT, DEG_RPT)], dst_v)
    pltpu.sync_copy(ew_hbm.at[pl.ds(w * DEG_RPT, DEG_RPT)], ew_v)

    def zb(r, carry):
        deg_v[pl.ds(r * 16, 16)] = jnp.zeros((16,), jnp.float32)
        return carry

    lax.fori_loop(0, N // 16, zb, 0)

    def gb(g, carry):
        for j in range(C // 16):
            sl = pl.ds(j * 16, 16)
            plsc.addupdate_scatter(deg_v, [dst_v[g, sl]], ew_v[g, sl])
        return carry

    lax.fori_loop(0, DEG_RPT, gb, 0)
    pltpu.sync_copy(deg_v, out_hbm.at[w])


def _make_agg(D):
    @functools.partial(
        pl.kernel,
        out_type=jax.ShapeDtypeStruct((NC, NPAD, D), jnp.float32),
        mesh=_mesh,
        compiler_params=_sc_params,
        scratch_types=[
            pltpu.VMEM((GMX, C), jnp.int32),
            pltpu.VMEM((GMX, C), jnp.int32),
            pltpu.VMEM((GMX, C), jnp.float32),
            pltpu.VMEM((N,), jnp.float32),
            pltpu.VMEM((GMX, C), jnp.float32),
            pltpu.VMEM((C, D), jnp.float32),
            pltpu.VMEM((C, D), jnp.float32),
            pltpu.VMEM((C, D), jnp.float32),
            pltpu.VMEM_SHARED((NPAD, D), jnp.float32),
            pltpu.SemaphoreType.DMA,
            pltpu.SemaphoreType.DMA,
            pltpu.SemaphoreType.DMA,
            pltpu.SemaphoreType.DMA,
            pltpu.SemaphoreType.DMA,
            pltpu.SemaphoreType.DMA,
        ],
    )
    def agg(src_hbm, dst_hbm, ew_hbm, dinv_hbm, y_hbm, out_hbm,
            src_v, dst_v, ew_v, dinv_v, norm_v, rows0, rows1, rows2, acc_sh,
            sg0, sg1, sg2, ss0, ss1, ss2):
        c = lax.axis_index("c")
        s = lax.axis_index("s")
        G = jnp.where(c == 0, G0, G1)
        start = jnp.where(c == 0, s * G0, NS * G0 + s * G1)
        st0 = pltpu.async_copy(src_hbm.at[pl.ds(start, GMX)], src_v, sg0)
        st1 = pltpu.async_copy(dst_hbm.at[pl.ds(start, GMX)], dst_v, sg1)
        st2 = pltpu.async_copy(ew_hbm.at[pl.ds(start, GMX)], ew_v, sg2)
        st3 = pltpu.async_copy(dinv_hbm, dinv_v, ss0)

        @plsc.parallel_loop(0, C, 1, unroll=2)
        def zb(r):
            for k in range(D // 16):
                rows0[r, pl.ds(k * 16, 16)] = jnp.zeros((16,), jnp.float32)

        for q in range(NPT // C):
            pltpu.async_copy(rows0, acc_sh.at[pl.ds(s * NPT + q * C, C)], ss1)
        for q in range(NPT // C):
            pltpu.make_async_copy(
                rows0, acc_sh.at[pl.ds(s * NPT, C)], ss1).wait()
        st0.wait()
        st1.wait()
        st2.wait()
        st3.wait()

        bufs = (rows0, rows1, rows2)
        gsems = (sg0, sg1, sg2)
        ssems = (ss0, ss1, ss2)

        def fire_gather(g, b, sg):
            pltpu.async_copy(y_hbm.at[src_v.at[g]], b, sg)

        def wait_gather(g, b, sg):
            pltpu.make_async_copy(y_hbm.at[src_v.at[g]], b, sg).wait()

        def fire_scatter(g, b, ss):
            pltpu.async_copy(b, acc_sh.at[dst_v.at[g]], ss, add=True)

        def wait_scatter(g, b, ss):
            pltpu.make_async_copy(b, acc_sh.at[dst_v.at[g]], ss).wait()

        def scale(g, b):
            @plsc.parallel_loop(0, C // 16, 1, unroll=2)
            def sb(j):
                nv = norm_v[g, pl.ds(j * 16, 16)]
                for l in range(16):
                    sc_ = nv[l]
                    e = j * 16 + l
                    for k in range(D // 16):
                        slk = pl.ds(k * 16, 16)
                        b[e, slk] = b[e, slk] * sc_

        fire_gather(0, rows0, sg0)
        fire_gather(1, rows1, sg1)

        @plsc.parallel_loop(0, GMX, 1, unroll=2)
        def nb(g):
            for j in range(C // 16):
                sl = pl.ds(j * 16, 16)
                nv = (ew_v[g, sl]
                      * plsc.load_gather(dinv_v, [src_v[g, sl]])
                      * plsc.load_gather(dinv_v, [dst_v[g, sl]]))
                norm_v[g, sl] = nv
        plsc.subcore_barrier()

        def gb(t, carry):
            g0 = 3 * t
            for i in range(3):
                g = g0 + i
                b, sg, ss = bufs[i], gsems[i], ssems[i]
                b2, sg2_, ss2_ = (bufs[(i + 2) % 3], gsems[(i + 2) % 3],
                                  ssems[(i + 2) % 3])
                wait_gather(g, b, sg)
                scale(g, b)

                @pl.when(g + 2 < G)
                def _():
                    @pl.when(g >= 1)
                    def __():
                        wait_scatter(g - 1, b2, ss2_)
                    fire_gather(g + 2, b2, sg2_)

                fire_scatter(g, b, ss)
            return carry

        lax.fori_loop(0, G // 3, gb, 0)
        wait_scatter(G - 3, rows0, ss0)
        wait_scatter(G - 2, rows1, ss1)
        wait_scatter(G - 1, rows2, ss2)
        plsc.subcore_barrier()

        obufs = (rows0, rows1)
        osems = (sg0, sg1)
        for q in range(NPT // C):
            b, so = obufs[q % 2], osems[q % 2]
            if q >= 2:
                pltpu.make_async_copy(
                    b, out_hbm.at[c, pl.ds(s * NPT, C)], so).wait()
            pltpu.sync_copy(acc_sh.at[pl.ds(s * NPT + q * C, C)], b)
            pltpu.async_copy(b, out_hbm.at[c, pl.ds(s * NPT + q * C, C)], so)
        for q in range(2):
            pltpu.make_async_copy(
                obufs[q], out_hbm.at[c, pl.ds(s * NPT, C)], osems[q]).wait()

    return agg


_agg_h = _make_agg(DH)
_agg_o = _make_agg(DOUTP)


def _mm1_body(x_ref, w_ref, dp_ref, o_ref, dinv_ref):
    o_ref[...] = jnp.dot(x_ref[...], w_ref[...],
                         preferred_element_type=jnp.float32)
    deg = jnp.sum(dp_ref[...], axis=0, keepdims=True)
    dinv_ref[...] = jnp.where(deg > 0,
                              lax.rsqrt(jnp.maximum(deg, 1e-12)),
                              jnp.zeros_like(deg))


def _mm2_body(p_ref, b_ref, w_ref, o_ref):
    h = p_ref[0] + p_ref[1] + b_ref[...]
    o_ref[...] = jnp.dot(h, w_ref[...], preferred_element_type=jnp.float32)


def _fin_body(p_ref, b_ref, o_ref):
    o_ref[...] = (p_ref[0] + p_ref[1] + b_ref[...])[:N, :DOUT]


def kernel(x, edge_index, edge_attr, W1, b1, W2, b2):
    loop = jnp.arange(N, dtype=jnp.int32)
    padi = jnp.zeros((EPAD - E2,), jnp.int32)
    padf = jnp.zeros((EPAD - E2,), jnp.float32)
    src = jnp.concatenate([edge_index[0].astype(jnp.int32), loop, padi])
    dst = jnp.concatenate([edge_index[1].astype(jnp.int32), loop, padi])
    ew = jnp.concatenate([edge_attr, jnp.ones((N,), jnp.float32), padf])
    srcg = src.reshape(NGT, C)
    dstg = dst.reshape(NGT, C)
    ewg = ew.reshape(NGT, C)

    deg_parts = _deg_sc(dstg, ewg)
    xw1, dinv = pl.pallas_call(
        _mm1_body,
        out_shape=(jax.ShapeDtypeStruct((N, DH), jnp.float32),
                   jax.ShapeDtypeStruct((1, N), jnp.float32)),
    )(x, W1, deg_parts)
    dinv = dinv.reshape(N)

    p1 = _agg_h(srcg, dstg, ewg, dinv, xw1)

    W2p = jnp.pad(W2, ((0, 0), (0, DOUTP - DOUT)))
    xw2 = pl.pallas_call(
        _mm2_body,
        out_shape=jax.ShapeDtypeStruct((NPAD, DOUTP), jnp.float32),
    )(p1, b1.reshape(1, DH), W2p)

    p2 = _agg_o(srcg, dstg, ewg, dinv, xw2)

    b2p = jnp.pad(b2, (0, DOUTP - DOUT)).reshape(1, DOUTP)
    out = pl.pallas_call(
        _fin_body,
        out_shape=jax.ShapeDtypeStruct((N, DOUT), jnp.float32),
    )(p2, b2p)
    return out


# speedup vs baseline: 1.1602x; 1.1602x over previous
import functools

import jax
import jax.numpy as jnp
from jax import lax
from jax.experimental import pallas as pl
from jax.experimental.pallas import tpu as pltpu
from jax.experimental.pallas import tpu_sc as plsc

N = 10000
E = 320000
DIN = 128
DH = 64
DOUT = 40
DOUTP = 48

NC = 2
NS = 16
NW = NC * NS
C = 128

E2 = E + N
G0 = 96
G1 = 72
GMX = max(G0, G1)
NG_AGG = NS * (G0 + G1)
DEG_RPT = 88
NGT = NW * DEG_RPT
EPAD = NGT * C
NPAD = 10240
NPT = NPAD // NS
assert E2 <= NG_AGG * C and NS * G0 + (NS - 1) * G1 + GMX <= NGT
assert G0 % 24 == 0 and G1 % 24 == 0 and DEG_RPT % 8 == 0
assert NPT % C == 0 and N % 16 == 0

_mesh = plsc.VectorSubcoreMesh(core_axis_name="c", subcore_axis_name="s")
_sc_params = pltpu.CompilerParams(needs_layout_passes=False,
                                  use_tc_tiling_on_sc=False)


@functools.partial(
    pl.kernel,
    out_type=jax.ShapeDtypeStruct((NW, N), jnp.float32),
    mesh=_mesh,
    compiler_params=_sc_params,
    scratch_types=[
        pltpu.VMEM((DEG_RPT, C), jnp.int32),
        pltpu.VMEM((DEG_RPT, C), jnp.float32),
        pltpu.VMEM((N,), jnp.float32),
    ],
)
def _deg_sc(dst_hbm, ew_hbm, out_hbm, dst_v, ew_v, deg_v):
    c = lax.axis_index("c")
    s = lax.axis_index("s")
    w = c * NS + s
    pltpu.sync_copy(dst_hbm.at[w], dst_v)
    pltpu.sync_copy(ew_hbm.at[w], ew_v)

    def zb(r, carry):
        deg_v[pl.ds(r * 16, 16)] = jnp.zeros((16,), jnp.float32)
        return carry

    lax.fori_loop(0, N // 16, zb, 0)

    def gb(g, carry):
        for j in range(C // 16):
            sl = pl.ds(j * 16, 16)
            plsc.addupdate_scatter(deg_v, [dst_v[g, sl]], ew_v[g, sl])
        return carry

    lax.fori_loop(0, DEG_RPT, gb, 0)
    pltpu.sync_copy(deg_v, out_hbm.at[w])


def _make_agg(D):
    @functools.partial(
        pl.kernel,
        out_type=jax.ShapeDtypeStruct((NC, NPAD, D), jnp.float32),
        mesh=_mesh,
        compiler_params=_sc_params,
        scratch_types=[
            pltpu.VMEM((GMX, C), jnp.int32),
            pltpu.VMEM((GMX, C), jnp.int32),
            pltpu.VMEM((GMX, C), jnp.float32),
            pltpu.VMEM((N,), jnp.float32),
            pltpu.VMEM((GMX, C), jnp.float32),
            pltpu.VMEM((C, D), jnp.float32),
            pltpu.VMEM((C, D), jnp.float32),
            pltpu.VMEM((C, D), jnp.float32),
            pltpu.VMEM_SHARED((NPAD, D), jnp.float32),
            pltpu.SemaphoreType.DMA,
            pltpu.SemaphoreType.DMA,
            pltpu.SemaphoreType.DMA,
            pltpu.SemaphoreType.DMA,
            pltpu.SemaphoreType.DMA,
            pltpu.SemaphoreType.DMA,
        ],
    )
    def agg(src_hbm, dst_hbm, ew_hbm, dinv_hbm, y_hbm, out_hbm,
            src_v, dst_v, ew_v, dinv_v, norm_v, rows0, rows1, rows2, acc_sh,
            sg0, sg1, sg2, ss0, ss1, ss2):
        c = lax.axis_index("c")
        s = lax.axis_index("s")
        w = c * NS + s
        G = jnp.where(c == 0, G0, G1)
        st0 = pltpu.async_copy(src_hbm.at[w], src_v, sg0)
        st1 = pltpu.async_copy(dst_hbm.at[w], dst_v, sg1)
        st2 = pltpu.async_copy(ew_hbm.at[w], ew_v, sg2)
        st3 = pltpu.async_copy(dinv_hbm, dinv_v, ss0)

        @plsc.parallel_loop(0, C, 1, unroll=2)
        def zb(r):
            for k in range(D // 16):
                rows0[r, pl.ds(k * 16, 16)] = jnp.zeros((16,), jnp.float32)

        for q in range(NPT // C):
            pltpu.async_copy(rows0, acc_sh.at[pl.ds(s * NPT + q * C, C)], ss1)
        for q in range(NPT // C):
            pltpu.make_async_copy(
                rows0, acc_sh.at[pl.ds(s * NPT, C)], ss1).wait()
        st0.wait()
        st1.wait()
        st2.wait()
        st3.wait()

        bufs = (rows0, rows1, rows2)
        gsems = (sg0, sg1, sg2)
        ssems = (ss0, ss1, ss2)

        def fire_gather(g, b, sg):
            pltpu.async_copy(y_hbm.at[src_v.at[g]], b, sg)

        def wait_gather(g, b, sg):
            pltpu.make_async_copy(y_hbm.at[src_v.at[g]], b, sg).wait()

        def fire_scatter(g, b, ss):
            pltpu.async_copy(b, acc_sh.at[dst_v.at[g]], ss, add=True)

        def wait_scatter(g, b, ss):
            pltpu.make_async_copy(b, acc_sh.at[dst_v.at[g]], ss).wait()

        def scale(g, b):
            @plsc.parallel_loop(0, C // 16, 1, unroll=2)
            def sb(j):
                nv = norm_v[g, pl.ds(j * 16, 16)]
                for l in range(16):
                    sc_ = nv[l]
                    e = j * 16 + l
                    for k in range(D // 16):
                        slk = pl.ds(k * 16, 16)
                        b[e, slk] = b[e, slk] * sc_

        fire_gather(0, rows0, sg0)
        fire_gather(1, rows1, sg1)

        @plsc.parallel_loop(0, GMX, 1, unroll=2)
        def nb(g):
            for j in range(C // 16):
                sl = pl.ds(j * 16, 16)
                nv = (ew_v[g, sl]
                      * plsc.load_gather(dinv_v, [src_v[g, sl]])
                      * plsc.load_gather(dinv_v, [dst_v[g, sl]]))
                norm_v[g, sl] = nv
        plsc.subcore_barrier()

        def gb(t, carry):
            g0 = 3 * t
            for i in range(3):
                g = g0 + i
                b, sg, ss = bufs[i], gsems[i], ssems[i]
                b2, sg2_, ss2_ = (bufs[(i + 2) % 3], gsems[(i + 2) % 3],
                                  ssems[(i + 2) % 3])
                wait_gather(g, b, sg)
                scale(g, b)

                @pl.when(g + 2 < G)
                def _():
                    @pl.when(g >= 1)
                    def __():
                        wait_scatter(g - 1, b2, ss2_)
                    fire_gather(g + 2, b2, sg2_)

                fire_scatter(g, b, ss)
            return carry

        lax.fori_loop(0, G // 3, gb, 0)
        wait_scatter(G - 3, rows0, ss0)
        wait_scatter(G - 2, rows1, ss1)
        wait_scatter(G - 1, rows2, ss2)
        plsc.subcore_barrier()

        obufs = (rows0, rows1)
        osems = (sg0, sg1)
        for q in range(NPT // C):
            b, so = obufs[q % 2], osems[q % 2]
            if q >= 2:
                pltpu.make_async_copy(
                    b, out_hbm.at[c, pl.ds(s * NPT, C)], so).wait()
            pltpu.sync_copy(acc_sh.at[pl.ds(s * NPT + q * C, C)], b)
            pltpu.async_copy(b, out_hbm.at[c, pl.ds(s * NPT + q * C, C)], so)
        for q in range(2):
            pltpu.make_async_copy(
                obufs[q], out_hbm.at[c, pl.ds(s * NPT, C)], osems[q]).wait()

    return agg


_agg_h = _make_agg(DH)
_agg_o = _make_agg(DOUTP)


def _mm1_body(x_ref, w_ref, dp_ref, o_ref, dinv_ref):
    o_ref[...] = jnp.dot(x_ref[...], w_ref[...],
                         preferred_element_type=jnp.float32)
    deg = jnp.sum(dp_ref[...], axis=0, keepdims=True)
    dinv_ref[...] = jnp.where(deg > 0,
                              lax.rsqrt(jnp.maximum(deg, 1e-12)),
                              jnp.zeros_like(deg))


def _mm2_body(p_ref, b_ref, w_ref, o_ref):
    h = p_ref[0] + p_ref[1] + b_ref[...]
    o_ref[...] = jnp.dot(h, w_ref[...], preferred_element_type=jnp.float32)


def _fin_body(p_ref, b_ref, o_ref):
    o_ref[...] = (p_ref[0] + p_ref[1] + b_ref[...])[:N, :DOUT]


def kernel(x, edge_index, edge_attr, W1, b1, W2, b2):
    loop = jnp.arange(N, dtype=jnp.int32)
    padi = jnp.zeros((EPAD - E2,), jnp.int32)
    padf = jnp.zeros((EPAD - E2,), jnp.float32)
    src = jnp.concatenate([edge_index[0].astype(jnp.int32), loop, padi])
    dst = jnp.concatenate([edge_index[1].astype(jnp.int32), loop, padi])
    ew = jnp.concatenate([edge_attr, jnp.ones((N,), jnp.float32), padf])
    def slots(a):
        flat = a.reshape(NGT, C)
        p0 = flat[:NS * G0].reshape(NS, G0, C)
        p1 = flat[NS * G0:NS * (G0 + G1)].reshape(NS, G1, C)
        p0 = jnp.pad(p0, ((0, 0), (0, GMX - G0), (0, 0)))
        p1 = jnp.pad(p1, ((0, 0), (0, GMX - G1), (0, 0)))
        return jnp.concatenate([p0, p1], axis=0)

    srcg = slots(src)
    dstg = slots(dst)
    ewg = slots(ew)
    dstd = dst.reshape(NW, DEG_RPT, C)
    ewd = ew.reshape(NW, DEG_RPT, C)

    deg_parts = _deg_sc(dstd, ewd)
    xw1, dinv = pl.pallas_call(
        _mm1_body,
        out_shape=(jax.ShapeDtypeStruct((N, DH), jnp.float32),
                   jax.ShapeDtypeStruct((1, N), jnp.float32)),
    )(x, W1, deg_parts)
    dinv = dinv.reshape(N)

    p1 = _agg_h(srcg, dstg, ewg, dinv, xw1)

    W2p = jnp.pad(W2, ((0, 0), (0, DOUTP - DOUT)))
    xw2 = pl.pallas_call(
        _mm2_body,
        out_shape=jax.ShapeDtypeStruct((NPAD, DOUTP), jnp.float32),
    )(p1, b1.reshape(1, DH), W2p)

    p2 = _agg_o(srcg, dstg, ewg, dinv, xw2)

    b2p = jnp.pad(b2, (0, DOUTP - DOUT)).reshape(1, DOUTP)
    out = pl.pallas_call(
        _fin_body,
        out_shape=jax.ShapeDtypeStruct((N, DOUT), jnp.float32),
    )(p2, b2p)
    return out


# speedup vs baseline: 1.1657x; 1.0048x over previous
import functools

import jax
import jax.numpy as jnp
from jax import lax
from jax.experimental import pallas as pl
from jax.experimental.pallas import tpu as pltpu
from jax.experimental.pallas import tpu_sc as plsc

N = 10000
E = 320000
DIN = 128
DH = 64
DOUT = 40
DOUTP = 48

NC = 2
NS = 16
NW = NC * NS
C = 128

E2 = E + N
G0 = 96
G1 = 72
GMX = max(G0, G1)
NG_AGG = NS * (G0 + G1)
DEG_RPT = 88
NGT = NW * DEG_RPT
EPAD = NGT * C
NPAD = 10240
NPT = NPAD // NS
assert E2 <= NG_AGG * C and NS * G0 + (NS - 1) * G1 + GMX <= NGT
assert G0 % 24 == 0 and G1 % 24 == 0 and DEG_RPT % 8 == 0
assert NPT % C == 0 and N % 16 == 0

_mesh = plsc.VectorSubcoreMesh(core_axis_name="c", subcore_axis_name="s")
_sc_params = pltpu.CompilerParams(needs_layout_passes=False,
                                  use_tc_tiling_on_sc=False)


@functools.partial(
    pl.kernel,
    out_type=jax.ShapeDtypeStruct((NW, N), jnp.float32),
    mesh=_mesh,
    compiler_params=_sc_params,
    scratch_types=[
        pltpu.VMEM((DEG_RPT, C), jnp.int32),
        pltpu.VMEM((DEG_RPT, C), jnp.float32),
        pltpu.VMEM((N,), jnp.float32),
    ],
)
def _deg_sc(dst_hbm, ew_hbm, out_hbm, dst_v, ew_v, deg_v):
    c = lax.axis_index("c")
    s = lax.axis_index("s")
    w = c * NS + s
    pltpu.sync_copy(dst_hbm.at[w], dst_v)
    pltpu.sync_copy(ew_hbm.at[w], ew_v)

    def zb(r, carry):
        deg_v[pl.ds(r * 16, 16)] = jnp.zeros((16,), jnp.float32)
        return carry

    lax.fori_loop(0, N // 16, zb, 0)

    def gb(g, carry):
        for j in range(C // 16):
            sl = pl.ds(j * 16, 16)
            plsc.addupdate_scatter(deg_v, [dst_v[g, sl]], ew_v[g, sl])
        return carry

    lax.fori_loop(0, DEG_RPT, gb, 0)
    pltpu.sync_copy(deg_v, out_hbm.at[w])


def _make_agg(D):
    @functools.partial(
        pl.kernel,
        out_type=jax.ShapeDtypeStruct((NC, NPAD, D), jnp.float32),
        mesh=_mesh,
        compiler_params=_sc_params,
        scratch_types=[
            pltpu.VMEM((GMX, C), jnp.int32),
            pltpu.VMEM((GMX, C), jnp.int32),
            pltpu.VMEM((GMX, C), jnp.float32),
            pltpu.VMEM((N,), jnp.float32),
            pltpu.VMEM((GMX, C), jnp.float32),
            pltpu.VMEM((C, D), jnp.float32),
            pltpu.VMEM((C, D), jnp.float32),
            pltpu.VMEM((C, D), jnp.float32),
            pltpu.VMEM_SHARED((NPAD, D), jnp.float32),
            pltpu.SemaphoreType.DMA,
            pltpu.SemaphoreType.DMA,
            pltpu.SemaphoreType.DMA,
            pltpu.SemaphoreType.DMA,
            pltpu.SemaphoreType.DMA,
            pltpu.SemaphoreType.DMA,
        ],
    )
    def agg(src_hbm, dst_hbm, ew_hbm, dinv_hbm, y_hbm, out_hbm,
            src_v, dst_v, ew_v, dinv_v, norm_v, rows0, rows1, rows2, acc_sh,
            sg0, sg1, sg2, ss0, ss1, ss2):
        c = lax.axis_index("c")
        s = lax.axis_index("s")
        w = c * NS + s
        st0 = pltpu.async_copy(src_hbm.at[w], src_v, sg0)
        st1 = pltpu.async_copy(dst_hbm.at[w], dst_v, sg1)
        st2 = pltpu.async_copy(ew_hbm.at[w], ew_v, sg2)
        st3 = pltpu.async_copy(dinv_hbm, dinv_v, ss0)

        @plsc.parallel_loop(0, C, 1, unroll=2)
        def zb(r):
            for k in range(D // 16):
                rows0[r, pl.ds(k * 16, 16)] = jnp.zeros((16,), jnp.float32)

        for q in range(NPT // C):
            pltpu.async_copy(rows0, acc_sh.at[pl.ds(s * NPT + q * C, C)], ss1)
        for q in range(NPT // C):
            pltpu.make_async_copy(
                rows0, acc_sh.at[pl.ds(s * NPT, C)], ss1).wait()
        st0.wait()
        st1.wait()
        st2.wait()
        st3.wait()

        bufs = (rows0, rows1, rows2)
        gsems = (sg0, sg1, sg2)
        ssems = (ss0, ss1, ss2)

        def fire_gather(g, b, sg):
            pltpu.async_copy(y_hbm.at[src_v.at[g]], b, sg)

        def wait_gather(g, b, sg):
            pltpu.make_async_copy(y_hbm.at[src_v.at[g]], b, sg).wait()

        def fire_scatter(g, b, ss):
            pltpu.async_copy(b, acc_sh.at[dst_v.at[g]], ss, add=True)

        def wait_scatter(g, b, ss):
            pltpu.make_async_copy(b, acc_sh.at[dst_v.at[g]], ss).wait()

        def scale(g, b):
            @plsc.parallel_loop(0, C // 16, 1, unroll=2)
            def sb(j):
                nv = norm_v[g, pl.ds(j * 16, 16)]
                for l in range(16):
                    sc_ = nv[l]
                    e = j * 16 + l
                    for k in range(D // 16):
                        slk = pl.ds(k * 16, 16)
                        b[e, slk] = b[e, slk] * sc_

        def run_edges(G):
            fire_gather(0, rows0, sg0)
            fire_gather(1, rows1, sg1)

            @plsc.parallel_loop(0, G, 1, unroll=2)
            def nb(g):
                for j in range(C // 16):
                    sl = pl.ds(j * 16, 16)
                    nv = (ew_v[g, sl]
                          * plsc.load_gather(dinv_v, [src_v[g, sl]])
                          * plsc.load_gather(dinv_v, [dst_v[g, sl]]))
                    norm_v[g, sl] = nv
            plsc.subcore_barrier()

            def gb(t, carry):
                g0 = 3 * t
                for i in range(3):
                    g = g0 + i
                    b, sg, ss = bufs[i], gsems[i], ssems[i]
                    b2, sg2_, ss2_ = (bufs[(i + 2) % 3], gsems[(i + 2) % 3],
                                      ssems[(i + 2) % 3])
                    wait_gather(g, b, sg)
                    scale(g, b)

                    @pl.when(g + 2 < G)
                    def _():
                        @pl.when(g >= 1)
                        def __():
                            wait_scatter(g - 1, b2, ss2_)
                        fire_gather(g + 2, b2, sg2_)

                    fire_scatter(g, b, ss)
                return carry

            lax.fori_loop(0, G // 3, gb, 0)
            wait_scatter(G - 3, rows0, ss0)
            wait_scatter(G - 2, rows1, ss1)
            wait_scatter(G - 1, rows2, ss2)
            plsc.subcore_barrier()

        @pl.when(c == 0)
        def _run0():
            run_edges(G0)

        @pl.when(c == 1)
        def _run1():
            run_edges(G1)

        obufs = (rows0, rows1)
        osems = (sg0, sg1)
        for q in range(NPT // C):
            b, so = obufs[q % 2], osems[q % 2]
            if q >= 2:
                pltpu.make_async_copy(
                    b, out_hbm.at[c, pl.ds(s * NPT, C)], so).wait()
            pltpu.sync_copy(acc_sh.at[pl.ds(s * NPT + q * C, C)], b)
            pltpu.async_copy(b, out_hbm.at[c, pl.ds(s * NPT + q * C, C)], so)
        for q in range(2):
            pltpu.make_async_copy(
                obufs[q], out_hbm.at[c, pl.ds(s * NPT, C)], osems[q]).wait()

    return agg


_agg_h = _make_agg(DH)
_agg_o = _make_agg(DOUTP)


def _mm1_body(x_ref, w_ref, dp_ref, o_ref, dinv_ref):
    o_ref[...] = jnp.dot(x_ref[...], w_ref[...],
                         preferred_element_type=jnp.float32)
    deg = jnp.sum(dp_ref[...], axis=0, keepdims=True)
    dinv_ref[...] = jnp.where(deg > 0,
                              lax.rsqrt(jnp.maximum(deg, 1e-12)),
                              jnp.zeros_like(deg))


def _mm2_body(p_ref, b_ref, w_ref, o_ref):
    h = p_ref[0] + p_ref[1] + b_ref[...]
    o_ref[...] = jnp.dot(h, w_ref[...], preferred_element_type=jnp.float32)


def _fin_body(p_ref, b_ref, o_ref):
    o_ref[...] = (p_ref[0] + p_ref[1] + b_ref[...])[:N, :DOUT]


def kernel(x, edge_index, edge_attr, W1, b1, W2, b2):
    loop = jnp.arange(N, dtype=jnp.int32)
    padi = jnp.zeros((EPAD - E2,), jnp.int32)
    padd = jnp.arange(EPAD - E2, dtype=jnp.int32) % N
    padf = jnp.zeros((EPAD - E2,), jnp.float32)
    src = jnp.concatenate([edge_index[0].astype(jnp.int32), loop, padi])
    dst = jnp.concatenate([edge_index[1].astype(jnp.int32), loop, padd])
    ew = jnp.concatenate([edge_attr, jnp.ones((N,), jnp.float32), padf])
    def slots(a):
        flat = a.reshape(NGT, C)
        p0 = flat[:NS * G0].reshape(NS, G0, C)
        p1 = flat[NS * G0:NS * (G0 + G1)].reshape(NS, G1, C)
        p0 = jnp.pad(p0, ((0, 0), (0, GMX - G0), (0, 0)))
        p1 = jnp.pad(p1, ((0, 0), (0, GMX - G1), (0, 0)))
        return jnp.concatenate([p0, p1], axis=0)

    srcg = slots(src)
    dstg = slots(dst)
    ewg = slots(ew)
    dstd = dst.reshape(NW, DEG_RPT, C)
    ewd = ew.reshape(NW, DEG_RPT, C)

    deg_parts = _deg_sc(dstd, ewd)
    xw1, dinv = pl.pallas_call(
        _mm1_body,
        out_shape=(jax.ShapeDtypeStruct((N, DH), jnp.float32),
                   jax.ShapeDtypeStruct((1, N), jnp.float32)),
    )(x, W1, deg_parts)
    dinv = dinv.reshape(N)

    p1 = _agg_h(srcg, dstg, ewg, dinv, xw1)

    W2p = jnp.pad(W2, ((0, 0), (0, DOUTP - DOUT)))
    xw2 = pl.pallas_call(
        _mm2_body,
        out_shape=jax.ShapeDtypeStruct((NPAD, DOUTP), jnp.float32),
    )(p1, b1.reshape(1, DH), W2p)

    p2 = _agg_o(srcg, dstg, ewg, dinv, xw2)

    b2p = jnp.pad(b2, (0, DOUTP - DOUT)).reshape(1, DOUTP)
    out = pl.pallas_call(
        _fin_body,
        out_shape=jax.ShapeDtypeStruct((N, DOUT), jnp.float32),
    )(p2, b2p)
    return out


# speedup vs baseline: 2.6923x; 2.3095x over previous
import functools

import jax
import jax.numpy as jnp
from jax import lax
from jax.experimental import pallas as pl
from jax.experimental.pallas import tpu as pltpu
from jax.experimental.pallas import tpu_sc as plsc

N = 10000
E = 320000
DIN = 128
DH = 64
DOUT = 40
DOUTP = 48

NC = 2
NS = 16
NW = NC * NS
C = 128

E2 = E + N
GPT = -(-E2 // (NW * C))
EPAD = NW * GPT * C
NGT = EPAD // C
NPAD = 10240
NPT = NPAD // NS
assert GPT % 3 == 0 and NPT % C == 0 and N % 16 == 0

_mesh = plsc.VectorSubcoreMesh(core_axis_name="c", subcore_axis_name="s")
_sc_params = pltpu.CompilerParams(needs_layout_passes=False,
                                  use_tc_tiling_on_sc=False)


@functools.partial(
    pl.kernel,
    out_type=jax.ShapeDtypeStruct((NW, N), jnp.float32),
    mesh=_mesh,
    compiler_params=_sc_params,
    scratch_types=[
        pltpu.VMEM((GPT, C), jnp.int32),
        pltpu.VMEM((GPT, C), jnp.float32),
        pltpu.VMEM((N,), jnp.float32),
    ],
)
def _deg_sc(dst_hbm, ew_hbm, out_hbm, dst_v, ew_v, deg_v):
    c = lax.axis_index("c")
    s = lax.axis_index("s")
    w = c * NS + s
    pltpu.sync_copy(dst_hbm.at[w], dst_v)
    pltpu.sync_copy(ew_hbm.at[w], ew_v)

    def zb(r, carry):
        deg_v[pl.ds(r * 16, 16)] = jnp.zeros((16,), jnp.float32)
        return carry

    lax.fori_loop(0, N // 16, zb, 0)

    def gb(g, carry):
        for j in range(C // 16):
            sl = pl.ds(j * 16, 16)
            plsc.addupdate_scatter(deg_v, [dst_v[g, sl]], ew_v[g, sl])
        return carry

    lax.fori_loop(0, GPT, gb, 0)
    pltpu.sync_copy(deg_v, out_hbm.at[w])


def _make_agg(D):
    @functools.partial(
        pl.kernel,
        out_type=jax.ShapeDtypeStruct((NC, NPAD, D), jnp.float32),
        mesh=_mesh,
        compiler_params=_sc_params,
        scratch_types=[
            pltpu.VMEM((GPT, C), jnp.int32),
            pltpu.VMEM((GPT, C), jnp.int32),
            pltpu.VMEM((GPT, C), jnp.float32),
            pltpu.VMEM((N,), jnp.float32),
            pltpu.VMEM((GPT, C), jnp.float32),
            pltpu.VMEM((C, D), jnp.float32),
            pltpu.VMEM((C, D), jnp.float32),
            pltpu.VMEM((C, D), jnp.float32),
            pltpu.VMEM_SHARED((NPAD, D), jnp.float32),
            pltpu.SemaphoreType.DMA,
            pltpu.SemaphoreType.DMA,
            pltpu.SemaphoreType.DMA,
            pltpu.SemaphoreType.DMA,
            pltpu.SemaphoreType.DMA,
            pltpu.SemaphoreType.DMA,
        ],
    )
    def agg(src_hbm, dst_hbm, ew_hbm, dinv_hbm, y_hbm, out_hbm,
            src_v, dst_v, ew_v, dinv_v, norm_v, rows0, rows1, rows2, acc_sh,
            sg0, sg1, sg2, ss0, ss1, ss2):
        c = lax.axis_index("c")
        s = lax.axis_index("s")
        w = c * NS + s
        st0 = pltpu.async_copy(src_hbm.at[w], src_v, sg0)
        st1 = pltpu.async_copy(dst_hbm.at[w], dst_v, sg1)
        st2 = pltpu.async_copy(ew_hbm.at[w], ew_v, sg2)
        st3 = pltpu.async_copy(dinv_hbm, dinv_v, ss0)

        @plsc.parallel_loop(0, C, 1, unroll=2)
        def zb(r):
            for k in range(D // 16):
                rows0[r, pl.ds(k * 16, 16)] = jnp.zeros((16,), jnp.float32)

        for q in range(NPT // C):
            pltpu.async_copy(rows0, acc_sh.at[pl.ds(s * NPT + q * C, C)], ss1)
        for q in range(NPT // C):
            pltpu.make_async_copy(
                rows0, acc_sh.at[pl.ds(s * NPT, C)], ss1).wait()
        st0.wait()
        st1.wait()
        st2.wait()
        st3.wait()

        bufs = (rows0, rows1, rows2)
        gsems = (sg0, sg1, sg2)
        ssems = (ss0, ss1, ss2)

        def fire_gather(g, b, sg):
            pltpu.async_copy(y_hbm.at[src_v.at[g]], b, sg)

        def wait_gather(g, b, sg):
            pltpu.make_async_copy(y_hbm.at[src_v.at[g]], b, sg).wait()

        def fire_scatter(g, b, ss):
            pltpu.async_copy(b, acc_sh.at[dst_v.at[g]], ss, add=True)

        def wait_scatter(g, b, ss):
            pltpu.make_async_copy(b, acc_sh.at[dst_v.at[g]], ss).wait()

        def scale(g, b):
            @plsc.parallel_loop(0, C // 16, 1, unroll=2)
            def sb(j):
                nv = norm_v[g, pl.ds(j * 16, 16)]
                for l in range(16):
                    sc_ = nv[l]
                    e = j * 16 + l
                    for k in range(D // 16):
                        slk = pl.ds(k * 16, 16)
                        b[e, slk] = b[e, slk] * sc_

        fire_gather(0, rows0, sg0)
        fire_gather(1, rows1, sg1)

        @plsc.parallel_loop(0, GPT, 1, unroll=2)
        def nb(g):
            for j in range(C // 16):
                sl = pl.ds(j * 16, 16)
                nv = (ew_v[g, sl]
                      * plsc.load_gather(dinv_v, [src_v[g, sl]])
                      * plsc.load_gather(dinv_v, [dst_v[g, sl]]))
                norm_v[g, sl] = nv
        plsc.subcore_barrier()

        def gb(t, carry):
            g0 = 3 * t
            for i in range(3):
                g = g0 + i
                b, sg, ss = bufs[i], gsems[i], ssems[i]
                b2, sg2_, ss2_ = (bufs[(i + 2) % 3], gsems[(i + 2) % 3],
                                  ssems[(i + 2) % 3])
                wait_gather(g, b, sg)
                scale(g, b)

                @pl.when(g + 2 < GPT)
                def _():
                    @pl.when(g >= 1)
                    def __():
                        wait_scatter(g - 1, b2, ss2_)
                    fire_gather(g + 2, b2, sg2_)

                fire_scatter(g, b, ss)
            return carry

        lax.fori_loop(0, GPT // 3, gb, 0)
        wait_scatter(GPT - 3, rows0, ss0)
        wait_scatter(GPT - 2, rows1, ss1)
        wait_scatter(GPT - 1, rows2, ss2)
        plsc.subcore_barrier()

        obufs = (rows0, rows1)
        osems = (sg0, sg1)
        for q in range(NPT // C):
            b, so = obufs[q % 2], osems[q % 2]
            if q >= 2:
                pltpu.make_async_copy(
                    b, out_hbm.at[c, pl.ds(s * NPT, C)], so).wait()
            pltpu.sync_copy(acc_sh.at[pl.ds(s * NPT + q * C, C)], b)
            pltpu.async_copy(b, out_hbm.at[c, pl.ds(s * NPT + q * C, C)], so)
        for q in range(2):
            pltpu.make_async_copy(
                obufs[q], out_hbm.at[c, pl.ds(s * NPT, C)], osems[q]).wait()

    return agg


_agg_h = _make_agg(DH)
_agg_o = _make_agg(DOUTP)


def _mm1_body(x_ref, w_ref, dp_ref, o_ref, dinv_ref):
    o_ref[...] = jnp.dot(x_ref[...], w_ref[...],
                         preferred_element_type=jnp.float32)
    deg = jnp.sum(dp_ref[...], axis=0, keepdims=True)
    dinv_ref[...] = jnp.where(deg > 0,
                              lax.rsqrt(jnp.maximum(deg, 1e-12)),
                              jnp.zeros_like(deg))


def _mm2_body(p_ref, b_ref, w_ref, o_ref):
    h = p_ref[0] + p_ref[1] + b_ref[...]
    o_ref[...] = jnp.dot(h, w_ref[...], preferred_element_type=jnp.float32)


def _fin_body(p_ref, b_ref, o_ref):
    o_ref[...] = (p_ref[0] + p_ref[1] + b_ref[...])[:N, :DOUT]


def kernel(x, edge_index, edge_attr, W1, b1, W2, b2):
    loop = jnp.arange(N, dtype=jnp.int32)
    padi = jnp.zeros((EPAD - E2,), jnp.int32)
    padd = jnp.arange(EPAD - E2, dtype=jnp.int32) % N
    padf = jnp.zeros((EPAD - E2,), jnp.float32)
    src = jnp.concatenate([edge_index[0].astype(jnp.int32), loop, padi])
    dst = jnp.concatenate([edge_index[1].astype(jnp.int32), loop, padd])
    ew = jnp.concatenate([edge_attr, jnp.ones((N,), jnp.float32), padf])
    srcg = src.reshape(NW, GPT, C)
    dstg = dst.reshape(NW, GPT, C)
    ewg = ew.reshape(NW, GPT, C)

    deg_parts = _deg_sc(dstg, ewg)
    xw1, dinv = pl.pallas_call(
        _mm1_body,
        out_shape=(jax.ShapeDtypeStruct((N, DH), jnp.float32),
                   jax.ShapeDtypeStruct((1, N), jnp.float32)),
    )(x, W1, deg_parts)
    dinv = dinv.reshape(N)

    p1 = _agg_h(srcg, dstg, ewg, dinv, xw1)

    W2p = jnp.pad(W2, ((0, 0), (0, DOUTP - DOUT)))
    xw2 = pl.pallas_call(
        _mm2_body,
        out_shape=jax.ShapeDtypeStruct((NPAD, DOUTP), jnp.float32),
    )(p1, b1.reshape(1, DH), W2p)

    p2 = _agg_o(srcg, dstg, ewg, dinv, xw2)

    b2p = jnp.pad(b2, (0, DOUTP - DOUT)).reshape(1, DOUTP)
    out = pl.pallas_call(
        _fin_body,
        out_shape=jax.ShapeDtypeStruct((N, DOUT), jnp.float32),
    )(p2, b2p)
    return out


# speedup vs baseline: 3.5507x; 1.3188x over previous
import functools

import jax
import jax.numpy as jnp
from jax import lax
from jax.experimental import pallas as pl
from jax.experimental.pallas import tpu as pltpu
from jax.experimental.pallas import tpu_sc as plsc

N = 10000
E = 320000
DIN = 128
DH = 64
DOUT = 40
DOUTP = 48

NC = 2
NS = 16
NW = NC * NS
C = 128

E2 = E + N
GPT = -(-E2 // (NW * C))
EPAD = NW * GPT * C
NGT = EPAD // C
NPAD = 10240
NPT = NPAD // NS
assert GPT % 3 == 0 and NPT % C == 0 and N % 16 == 0

_mesh = plsc.VectorSubcoreMesh(core_axis_name="c", subcore_axis_name="s")
_sc_params = pltpu.CompilerParams(needs_layout_passes=False,
                                  use_tc_tiling_on_sc=False)


@functools.partial(
    pl.kernel,
    out_type=jax.ShapeDtypeStruct((NW, N), jnp.float32),
    mesh=_mesh,
    compiler_params=_sc_params,
    scratch_types=[
        pltpu.VMEM((GPT, C), jnp.int32),
        pltpu.VMEM((GPT, C), jnp.float32),
        pltpu.VMEM((N,), jnp.float32),
    ],
)
def _deg_sc(dst_hbm, ew_hbm, out_hbm, dst_v, ew_v, deg_v):
    c = lax.axis_index("c")
    s = lax.axis_index("s")
    w = c * NS + s
    pltpu.sync_copy(dst_hbm.at[w], dst_v)
    pltpu.sync_copy(ew_hbm.at[w], ew_v)

    def zb(r, carry):
        deg_v[pl.ds(r * 16, 16)] = jnp.zeros((16,), jnp.float32)
        return carry

    lax.fori_loop(0, N // 16, zb, 0)

    def gb(g, carry):
        for j in range(C // 16):
            sl = pl.ds(j * 16, 16)
            plsc.addupdate_scatter(deg_v, [dst_v[g, sl]], ew_v[g, sl])
        return carry

    lax.fori_loop(0, GPT, gb, 0)
    pltpu.sync_copy(deg_v, out_hbm.at[w])


def _make_agg(D):
    @functools.partial(
        pl.kernel,
        out_type=jax.ShapeDtypeStruct((NC, NPAD, D), jnp.float32),
        mesh=_mesh,
        compiler_params=_sc_params,
        scratch_types=[
            pltpu.VMEM((GPT, C), jnp.int32),
            pltpu.VMEM((GPT, C), jnp.int32),
            pltpu.VMEM((GPT, C), jnp.float32),
            pltpu.VMEM((N,), jnp.float32),
            pltpu.VMEM((GPT, C), jnp.float32),
            pltpu.VMEM((C, D), jnp.float32),
            pltpu.VMEM((C, D), jnp.float32),
            pltpu.VMEM((C, D), jnp.float32),
            pltpu.VMEM_SHARED((NPAD, D), jnp.float32),
            pltpu.SemaphoreType.DMA,
            pltpu.SemaphoreType.DMA,
            pltpu.SemaphoreType.DMA,
            pltpu.SemaphoreType.DMA,
            pltpu.SemaphoreType.DMA,
            pltpu.SemaphoreType.DMA,
        ],
    )
    def agg(src_hbm, dst_hbm, ew_hbm, dinv_hbm, y_hbm, out_hbm,
            src_v, dst_v, ew_v, dinv_v, norm_v, rows0, rows1, rows2, acc_sh,
            sg0, sg1, sg2, ss0, ss1, ss2):
        c = lax.axis_index("c")
        s = lax.axis_index("s")
        w = c * NS + s
        st0 = pltpu.async_copy(src_hbm.at[w], src_v, sg0)
        st1 = pltpu.async_copy(dst_hbm.at[w], dst_v, sg1)
        st2 = pltpu.async_copy(ew_hbm.at[w], ew_v, sg2)
        st3 = pltpu.async_copy(dinv_hbm, dinv_v, ss0)

        @plsc.parallel_loop(0, C, 1, unroll=2)
        def zb(r):
            for k in range(D // 16):
                rows0[r, pl.ds(k * 16, 16)] = jnp.zeros((16,), jnp.float32)

        for q in range(NPT // C):
            pltpu.async_copy(rows0, acc_sh.at[pl.ds(s * NPT + q * C, C)], ss1)
        for q in range(NPT // C):
            pltpu.make_async_copy(
                rows0, acc_sh.at[pl.ds(s * NPT, C)], ss1).wait()
        st0.wait()
        st1.wait()
        st2.wait()
        st3.wait()

        bufs = (rows0, rows1, rows2)
        gsems = (sg0, sg1, sg2)
        ssems = (ss0, ss1, ss2)

        def fire_gather(g, b, sg):
            pltpu.async_copy(y_hbm.at[src_v.at[g]], b, sg)

        def wait_gather(g, b, sg):
            pltpu.make_async_copy(y_hbm.at[src_v.at[g]], b, sg).wait()

        def fire_scatter(g, b, ss):
            pltpu.async_copy(b, acc_sh.at[dst_v.at[g]], ss, add=True)

        def wait_scatter(g, b, ss):
            pltpu.make_async_copy(b, acc_sh.at[dst_v.at[g]], ss).wait()

        def scale(g, b):
            @plsc.parallel_loop(0, C // 16, 1, unroll=2)
            def sb(j):
                nv = norm_v[g, pl.ds(j * 16, 16)]
                for l in range(16):
                    sc_ = nv[l]
                    e = j * 16 + l
                    for k in range(D // 16):
                        slk = pl.ds(k * 16, 16)
                        b[e, slk] = b[e, slk] * sc_

        fire_gather(0, rows0, sg0)
        fire_gather(1, rows1, sg1)

        @plsc.parallel_loop(0, GPT, 1, unroll=2)
        def nb(g):
            for j in range(C // 16):
                sl = pl.ds(j * 16, 16)
                nv = (ew_v[g, sl]
                      * plsc.load_gather(dinv_v, [src_v[g, sl]])
                      * plsc.load_gather(dinv_v, [dst_v[g, sl]]))
                norm_v[g, sl] = nv
        plsc.subcore_barrier()

        def gb(t, carry):
            g0 = 3 * t
            for i in range(3):
                g = g0 + i
                b, sg, ss = bufs[i], gsems[i], ssems[i]
                b2, sg2_, ss2_ = (bufs[(i + 2) % 3], gsems[(i + 2) % 3],
                                  ssems[(i + 2) % 3])
                wait_gather(g, b, sg)
                scale(g, b)

                @pl.when(g + 2 < GPT)
                def _():
                    @pl.when(g >= 1)
                    def __():
                        wait_scatter(g - 1, b2, ss2_)
                    fire_gather(g + 2, b2, sg2_)

                fire_scatter(g, b, ss)
            return carry

        lax.fori_loop(0, GPT // 3, gb, 0)
        wait_scatter(GPT - 3, rows0, ss0)
        wait_scatter(GPT - 2, rows1, ss1)
        wait_scatter(GPT - 1, rows2, ss2)
        plsc.subcore_barrier()

        obufs = (rows0, rows1)
        osems = (sg0, sg1)
        for q in range(NPT // C):
            b, so = obufs[q % 2], osems[q % 2]
            if q >= 2:
                pltpu.make_async_copy(
                    b, out_hbm.at[c, pl.ds(s * NPT, C)], so).wait()
            pltpu.sync_copy(acc_sh.at[pl.ds(s * NPT + q * C, C)], b)
            pltpu.async_copy(b, out_hbm.at[c, pl.ds(s * NPT + q * C, C)], so)
        for q in range(2):
            pltpu.make_async_copy(
                obufs[q], out_hbm.at[c, pl.ds(s * NPT, C)], osems[q]).wait()

    return agg


_agg_h = _make_agg(DH)
_agg_o = _make_agg(DOUTP)


def _mm1_body(x_ref, w_ref, dp_ref, o_ref, dinv_ref):
    o_ref[...] = jnp.dot(x_ref[...], w_ref[...],
                         preferred_element_type=jnp.float32)
    deg = jnp.sum(dp_ref[...], axis=0, keepdims=True)
    dinv_ref[...] = jnp.where(deg > 0,
                              lax.rsqrt(jnp.maximum(deg, 1e-12)),
                              jnp.zeros_like(deg))


def _mm2_body(p_ref, b_ref, w_ref, o_ref):
    h = p_ref[0] + p_ref[1] + b_ref[...]
    o_ref[...] = jnp.dot(h, w_ref[...], preferred_element_type=jnp.float32)


def _fin_body(p_ref, b_ref, o_ref):
    o_ref[...] = (p_ref[0] + p_ref[1] + b_ref[...])[:N, :DOUT]


def kernel(x, edge_index, edge_attr, W1, b1, W2, b2):
    loop = jnp.arange(N, dtype=jnp.int32)
    padd = jnp.arange(EPAD - E2, dtype=jnp.int32) % N
    padf = jnp.zeros((EPAD - E2,), jnp.float32)
    src = jnp.concatenate([edge_index[0].astype(jnp.int32), loop, padd])
    dst = jnp.concatenate([edge_index[1].astype(jnp.int32), loop, padd])
    ew = jnp.concatenate([edge_attr, jnp.ones((N,), jnp.float32), padf])
    srcg = src.reshape(NW, GPT, C)
    dstg = dst.reshape(NW, GPT, C)
    ewg = ew.reshape(NW, GPT, C)

    deg_parts = _deg_sc(dstg, ewg)
    xw1, dinv = pl.pallas_call(
        _mm1_body,
        out_shape=(jax.ShapeDtypeStruct((N, DH), jnp.float32),
                   jax.ShapeDtypeStruct((1, N), jnp.float32)),
    )(x, W1, deg_parts)
    dinv = dinv.reshape(N)

    p1 = _agg_h(srcg, dstg, ewg, dinv, xw1)

    W2p = jnp.pad(W2, ((0, 0), (0, DOUTP - DOUT)))
    xw2 = pl.pallas_call(
        _mm2_body,
        out_shape=jax.ShapeDtypeStruct((NPAD, DOUTP), jnp.float32),
    )(p1, b1.reshape(1, DH), W2p)

    p2 = _agg_o(srcg, dstg, ewg, dinv, xw2)

    b2p = jnp.pad(b2, (0, DOUTP - DOUT)).reshape(1, DOUTP)
    out = pl.pallas_call(
        _fin_body,
        out_shape=jax.ShapeDtypeStruct((N, DOUT), jnp.float32),
    )(p2, b2p)
    return out
